# baseline (speedup 1.0000x reference)
import functools

import jax
import jax.numpy as jnp
import numpy as np
from jax import lax
from jax.experimental import pallas as pl
from jax.experimental.pallas import tpu as pltpu
from jax.experimental.pallas import tpu_sc as plsc

N = 10000
E = 320000
NINP = 128
NE = 4
NOUT = 64
MW = 128
NG = 64
NC = 2
NS = 16
NW = NC * NS
C = 64
EW = 9984
NCHUNK = EW // C
NCHUNK_LAST = (E - (NW - 1) * EW) // C
ROWS_PER_TILE = 624
ROWS_TAIL = N - NS * ROWS_PER_TILE
POOL_STEP = 312
POOL_BUF = N - (NW - 1) * POOL_STEP


def _mm_kernel(x_ref, w_ref, o_ref):
    o_ref[...] = jnp.dot(x_ref[...], w_ref[...],
                         preferred_element_type=jnp.float32)


def _tc_matmul(x, w, bn):
    n, k = x.shape
    m = w.shape[1]
    return pl.pallas_call(
        _mm_kernel,
        grid=(n // bn,),
        in_specs=[pl.BlockSpec((bn, k), lambda i: (i, 0)),
                  pl.BlockSpec((k, m), lambda i: (0, 0))],
        out_specs=pl.BlockSpec((bn, m), lambda i: (i, 0)),
        out_shape=jax.ShapeDtypeStruct((n, m), jnp.float32),
    )(x, w)


def _comb_mm_kernel(p0_ref, p1_ref, b_ref, w_ref, o_ref):
    h = jnp.maximum(p0_ref[:, :NOUT] + p1_ref[:, :NOUT] + b_ref[...], 0.0)
    o_ref[...] = jnp.dot(h, w_ref[...], preferred_element_type=jnp.float32)


def _tc_combine_matmul(p0, p1, b, w, bn):
    n = p0.shape[0]
    k = NOUT
    m = w.shape[1]
    return pl.pallas_call(
        _comb_mm_kernel,
        grid=(n // bn,),
        in_specs=[pl.BlockSpec((bn, MW), lambda i: (i, 0)),
                  pl.BlockSpec((bn, MW), lambda i: (i, 0)),
                  pl.BlockSpec((1, k), lambda i: (0, 0)),
                  pl.BlockSpec((k, m), lambda i: (0, 0))],
        out_specs=pl.BlockSpec((bn, m), lambda i: (i, 0)),
        out_shape=jax.ShapeDtypeStruct((n, m), jnp.float32),
    )(p0, p1, b.reshape(1, k), w)


def _sc_edge_body(y_hbm, ei_hbm, attr_hbm, zeros_hbm, out_hbm,
                  srcv0, srcv1, dstv0, dstv1, attrv0, attrv1,
                  gbuf0, gbuf1, mbuf, acc, semi0, semi1, semg0, semg1):
    cid = lax.axis_index("c")
    sid = lax.axis_index("s")
    wid = sid * NC + cid
    srcv = (srcv0, srcv1)
    dstv = (dstv0, dstv1)
    attrv = (attrv0, attrv1)
    gbuf = (gbuf0, gbuf1)
    semi = (semi0, semi1)
    semg = (semg0, semg1)

    rsl = pl.ds(sid * ROWS_PER_TILE, ROWS_PER_TILE)
    tsl = pl.ds(NS * ROWS_PER_TILE, ROWS_TAIL)
    pltpu.sync_copy(zeros_hbm.at[rsl], acc.at[rsl])

    @pl.when(sid == NS - 1)
    def _():
        pltpu.sync_copy(zeros_hbm.at[tsl], acc.at[tsl])

    plsc.subcore_barrier()

    ebase = wid * EW
    nchunk = jnp.where(wid == NW - 1, NCHUNK_LAST, NCHUNK)

    def fire_idx(ci, b):
        base = ebase + ci * C
        pltpu.async_copy(ei_hbm.at[0, pl.ds(base, C)], srcv[b], semi[b])
        pltpu.async_copy(ei_hbm.at[1, pl.ds(base, C)], dstv[b], semi[b])
        for i in range(NE):
            pltpu.async_copy(attr_hbm.at[i, pl.ds(base, C)], attrv[b].at[i],
                             semi[b])

    def wait_idx(ci, b):
        base = ebase + ci * C
        pltpu.make_async_copy(ei_hbm.at[0, pl.ds(base, C)], srcv[b],
                              semi[b]).wait()
        pltpu.make_async_copy(ei_hbm.at[1, pl.ds(base, C)], dstv[b],
                              semi[b]).wait()
        for i in range(NE):
            pltpu.make_async_copy(attr_hbm.at[i, pl.ds(base, C)],
                                  attrv[b].at[i], semi[b]).wait()

    def fire_gather(b):
        pltpu.async_copy(y_hbm.at[srcv[b]], gbuf[b], semg[b])

    def wait_gather(b):
        pltpu.make_async_copy(y_hbm.at[srcv[b]], gbuf[b], semg[b]).wait()

    def compute_scatter(b):
        g = gbuf[b]
        a = attrv[b]

        @plsc.parallel_loop(0, C // 16, unroll=2)
        def _(q):
            sl = pl.ds(q * 16, 16)
            a0v = a[0, sl]
            a1v = a[1, sl]
            a2v = a[2, sl]
            a3v = a[3, sl]
            for k in range(16):
                e = q * 16 + k
                for j in range(NOUT // 16):
                    v = (g[e, pl.ds(j * 16, 16)] * a0v[k]
                         + g[e, pl.ds(64 + j * 16, 16)] * a1v[k]
                         + g[e, pl.ds(128 + j * 16, 16)] * a2v[k]
                         + g[e, pl.ds(192 + j * 16, 16)] * a3v[k])
                    mbuf[e, pl.ds(j * 16, 16)] = v

        pltpu.sync_copy(mbuf, acc.at[dstv[b]], add=True)

    fire_idx(0, 0)
    wait_idx(0, 0)
    fire_gather(0)
    fire_idx(1, 1)

    def pair(q, _):
        for b in range(2):
            ci = 2 * q + b

            @pl.when(ci < nchunk)
            def _():
                wait_gather(b)

                @pl.when(ci + 1 < nchunk)
                def _():
                    wait_idx(ci + 1, 1 - b)
                    fire_gather(1 - b)

                compute_scatter(b)

                @pl.when(ci + 2 < nchunk)
                def _():
                    fire_idx(ci + 2, b)

        return 0

    lax.fori_loop(0, (NCHUNK_LAST + 1) // 2, pair, 0)

    plsc.subcore_barrier()
    pltpu.sync_copy(acc.at[rsl], out_hbm.at[cid, rsl])

    @pl.when(sid == NS - 1)
    def _():
        pltpu.sync_copy(acc.at[tsl], out_hbm.at[cid, tsl])


def _sc_edge_pass(y, ei, attr_t, zeros):
    mesh = plsc.VectorSubcoreMesh(core_axis_name="c", subcore_axis_name="s",
                                  num_cores=NC, num_subcores=NS)
    f = pl.kernel(
        _sc_edge_body,
        out_type=jax.ShapeDtypeStruct((NC, N, MW), jnp.float32),
        mesh=mesh,
        scratch_types=[
            pltpu.VMEM((C,), jnp.int32),
            pltpu.VMEM((C,), jnp.int32),
            pltpu.VMEM((C,), jnp.int32),
            pltpu.VMEM((C,), jnp.int32),
            pltpu.VMEM((NE, C), jnp.float32),
            pltpu.VMEM((NE, C), jnp.float32),
            pltpu.VMEM((C, NE * NOUT), jnp.float32),
            pltpu.VMEM((C, NE * NOUT), jnp.float32),
            pltpu.VMEM((C, MW), jnp.float32),
            pltpu.VMEM_SHARED((N, MW), jnp.float32),
            pltpu.SemaphoreType.DMA,
            pltpu.SemaphoreType.DMA,
            pltpu.SemaphoreType.DMA,
            pltpu.SemaphoreType.DMA,
        ],
    )
    return f(y, ei, attr_t, zeros)


def _sc_pool_body(p0_hbm, p1_hbm, b_hbm, batch_hbm, z64_hbm, ninf_hbm,
                  psum_hbm, pmax_hbm, pcnt_hbm,
                  h0, h1, bb, bv, sacc, macc, cacc):
    cid = lax.axis_index("c")
    sid = lax.axis_index("s")
    wid = sid * NC + cid

    base = wid * POOL_STEP
    nsl = pl.ds(base, POOL_BUF)
    pltpu.sync_copy(p0_hbm.at[nsl], h0)
    pltpu.sync_copy(p1_hbm.at[nsl], h1)
    pltpu.sync_copy(batch_hbm.at[nsl], bb.at[pl.ds(0, POOL_BUF)])
    pltpu.sync_copy(b_hbm, bv)
    pltpu.sync_copy(z64_hbm, sacc)
    pltpu.sync_copy(ninf_hbm, macc)
    pltpu.sync_copy(z64_hbm.at[0], cacc)

    nrows = jnp.where(wid == NW - 1, POOL_BUF, POOL_STEP)

    def node(i, _):
        b = bb[pl.ds(i, 16)][0]
        for j in range(NOUT // 16):
            sl = pl.ds(j * 16, 16)
            h = jnp.maximum(h0[i, sl] + h1[i, sl] + bv[0, sl], 0.0)
            sacc[b, sl] = sacc[b, sl] + h
            macc[b, sl] = jnp.maximum(macc[b, sl], h)
        bq = b // 16
        br = b - bq * 16
        lane = lax.iota(jnp.int32, 16)
        csl = pl.ds(bq * 16, 16)
        cacc[csl] = cacc[csl] + jnp.where(lane == br, 1.0, 0.0)
        return 0

    lax.fori_loop(0, nrows, node, 0)
    pltpu.sync_copy(sacc, psum_hbm.at[wid])
    pltpu.sync_copy(macc, pmax_hbm.at[wid])
    pltpu.sync_copy(cacc, pcnt_hbm.at[wid])


def _sc_pool(p0, p1, b2, batch, z64, ninf):
    mesh = plsc.VectorSubcoreMesh(core_axis_name="c", subcore_axis_name="s",
                                  num_cores=NC, num_subcores=NS)
    f = pl.kernel(
        _sc_pool_body,
        out_type=(jax.ShapeDtypeStruct((NW, NG, NOUT), jnp.float32),
                  jax.ShapeDtypeStruct((NW, NG, NOUT), jnp.float32),
                  jax.ShapeDtypeStruct((NW, NG), jnp.float32)),
        mesh=mesh,
        scratch_types=[
            pltpu.VMEM((POOL_BUF, MW), jnp.float32),
            pltpu.VMEM((POOL_BUF, MW), jnp.float32),
            pltpu.VMEM((POOL_BUF + 16,), jnp.int32),
            pltpu.VMEM((1, NOUT), jnp.float32),
            pltpu.VMEM((NG, NOUT), jnp.float32),
            pltpu.VMEM((NG, NOUT), jnp.float32),
            pltpu.VMEM((NG,), jnp.float32),
        ],
    )
    return f(p0, p1, b2.reshape(1, NOUT), batch, z64, ninf)


def _final_kernel(ps_ref, pm_ref, pc_ref, wfc_ref, bfc_ref, o_ref):
    s = jnp.sum(ps_ref[...], axis=0)
    mx = jnp.max(pm_ref[...], axis=0)
    cnt = jnp.sum(pc_ref[...], axis=0)
    mean = s / jnp.maximum(cnt, 1.0)[:, None]
    g = jnp.concatenate([mean, mx], axis=1)
    logits = jnp.dot(g, wfc_ref[...], preferred_element_type=jnp.float32)
    logits = logits + bfc_ref[...]
    m = jnp.max(logits, axis=1, keepdims=True)
    lse = m + jnp.log(jnp.sum(jnp.exp(logits - m), axis=1, keepdims=True))
    o_ref[...] = logits - lse


def _tc_final(psum, pmax, pcnt, wfc, bfc):
    return pl.pallas_call(
        _final_kernel,
        in_specs=[pl.BlockSpec((NW, NG, NOUT), lambda: (0, 0, 0)),
                  pl.BlockSpec((NW, NG, NOUT), lambda: (0, 0, 0)),
                  pl.BlockSpec((NW, NG), lambda: (0, 0)),
                  pl.BlockSpec((2 * NOUT, 2), lambda: (0, 0)),
                  pl.BlockSpec((1, 2), lambda: (0, 0))],
        out_specs=pl.BlockSpec((NG, 2), lambda: (0, 0)),
        out_shape=jax.ShapeDtypeStruct((NG, 2), jnp.float32),
    )(psum, pmax, pcnt, wfc, bfc.reshape(1, 2))


def kernel(x, edge_index, edge_attr, batch, W1, b1, W2, b2, Wfc, bfc):
    w1c = jnp.transpose(W1, (1, 0, 2)).reshape(NINP, NE * NOUT)
    w2c = jnp.transpose(W2, (1, 0, 2)).reshape(NOUT, NE * NOUT)
    zeros = jnp.zeros((N, MW), jnp.float32)
    ninf = jnp.full((NG, NOUT), -jnp.inf, jnp.float32)
    z64 = jnp.zeros((NG, NOUT), jnp.float32)

    attr_t = edge_attr.T
    y1 = _tc_matmul(x, w1c, 2000)
    p1 = _sc_edge_pass(y1, edge_index, attr_t, zeros)
    y2 = _tc_combine_matmul(p1[0], p1[1], b1, w2c, 2000)
    p2 = _sc_edge_pass(y2, edge_index, attr_t, zeros)
    psum, pmax, pcnt = _sc_pool(p2[0], p2[1], b2, batch, z64, ninf)
    return _tc_final(psum, pmax, pcnt, Wfc, bfc)


# speedup vs baseline: 1.3956x; 1.3956x over previous
import functools

import jax
import jax.numpy as jnp
import numpy as np
from jax import lax
from jax.experimental import pallas as pl
from jax.experimental.pallas import tpu as pltpu
from jax.experimental.pallas import tpu_sc as plsc

N = 10000
E = 320000
NINP = 128
NE = 4
NOUT = 64
MW = 128
NG = 64
NC = 2
NS = 16
NW = NC * NS
C = 64
EW = 9984
NCHUNK = EW // C
NCHUNK_LAST = (E - (NW - 1) * EW) // C
ROWS_PER_TILE = 624
ROWS_TAIL = N - NS * ROWS_PER_TILE
POOL_STEP = 312
POOL_BUF = N - (NW - 1) * POOL_STEP


def _mm_kernel(x_ref, w_ref, o_ref):
    o_ref[...] = jnp.dot(x_ref[...], w_ref[...],
                         preferred_element_type=jnp.float32)


def _tc_matmul(x, w, bn):
    n, k = x.shape
    m = w.shape[1]
    return pl.pallas_call(
        _mm_kernel,
        grid=(n // bn,),
        in_specs=[pl.BlockSpec((bn, k), lambda i: (i, 0)),
                  pl.BlockSpec((k, m), lambda i: (0, 0))],
        out_specs=pl.BlockSpec((bn, m), lambda i: (i, 0)),
        out_shape=jax.ShapeDtypeStruct((n, m), jnp.float32),
    )(x, w)


def _comb_mm_kernel(p0_ref, p1_ref, b_ref, w_ref, o_ref):
    h = jnp.maximum(p0_ref[:, :NOUT] + p1_ref[:, :NOUT] + b_ref[...], 0.0)
    o_ref[...] = jnp.dot(h, w_ref[...], preferred_element_type=jnp.float32)


def _tc_combine_matmul(p0, p1, b, w, bn):
    n = p0.shape[0]
    k = NOUT
    m = w.shape[1]
    return pl.pallas_call(
        _comb_mm_kernel,
        grid=(n // bn,),
        in_specs=[pl.BlockSpec((bn, MW), lambda i: (i, 0)),
                  pl.BlockSpec((bn, MW), lambda i: (i, 0)),
                  pl.BlockSpec((1, k), lambda i: (0, 0)),
                  pl.BlockSpec((k, m), lambda i: (0, 0))],
        out_specs=pl.BlockSpec((bn, m), lambda i: (i, 0)),
        out_shape=jax.ShapeDtypeStruct((n, m), jnp.float32),
    )(p0, p1, b.reshape(1, k), w)


def _sc_edge_body(y_hbm, ei_hbm, attr_hbm, zeros_hbm, out_hbm,
                  srcv0, srcv1, dstv0, dstv1, attrv0, attrv1,
                  gbuf0, gbuf1, mbuf, acc, semi0, semi1, semg0, semg1):
    cid = lax.axis_index("c")
    sid = lax.axis_index("s")
    wid = sid * NC + cid
    srcv = (srcv0, srcv1)
    dstv = (dstv0, dstv1)
    attrv = (attrv0, attrv1)
    gbuf = (gbuf0, gbuf1)
    semi = (semi0, semi1)
    semg = (semg0, semg1)

    rsl = pl.ds(sid * ROWS_PER_TILE, ROWS_PER_TILE)
    tsl = pl.ds(NS * ROWS_PER_TILE, ROWS_TAIL)
    pltpu.sync_copy(zeros_hbm.at[rsl], acc.at[rsl])

    @pl.when(sid == NS - 1)
    def _():
        pltpu.sync_copy(zeros_hbm.at[tsl], acc.at[tsl])

    plsc.subcore_barrier()

    ebase = wid * EW
    nchunk = jnp.where(wid == NW - 1, NCHUNK_LAST, NCHUNK)

    def fire_idx(ci, b):
        base = ebase + ci * C
        pltpu.async_copy(ei_hbm.at[pl.ds(base, C)], srcv[b], semi[b])
        pltpu.async_copy(ei_hbm.at[pl.ds(E + base, C)], dstv[b], semi[b])
        pltpu.async_copy(attr_hbm.at[pl.ds(base * NE, C * NE)], attrv[b],
                         semi[b])

    def wait_idx(ci, b):
        base = ebase + ci * C
        pltpu.make_async_copy(ei_hbm.at[pl.ds(base, C)], srcv[b],
                              semi[b]).wait()
        pltpu.make_async_copy(ei_hbm.at[pl.ds(E + base, C)], dstv[b],
                              semi[b]).wait()
        pltpu.make_async_copy(attr_hbm.at[pl.ds(base * NE, C * NE)], attrv[b],
                              semi[b]).wait()

    def fire_gather(b):
        pltpu.async_copy(y_hbm.at[srcv[b]], gbuf[b], semg[b])

    def wait_gather(b):
        pltpu.make_async_copy(y_hbm.at[srcv[b]], gbuf[b], semg[b]).wait()

    def compute_scatter(b):
        g = gbuf[b]
        a = attrv[b]

        @plsc.parallel_loop(0, C // 4, unroll=2)
        def _(q):
            av = a[pl.ds(q * 16, 16)]
            for k in range(4):
                e = q * 4 + k
                for j in range(NOUT // 16):
                    v = (g[e, pl.ds(j * 16, 16)] * av[4 * k]
                         + g[e, pl.ds(64 + j * 16, 16)] * av[4 * k + 1]
                         + g[e, pl.ds(128 + j * 16, 16)] * av[4 * k + 2]
                         + g[e, pl.ds(192 + j * 16, 16)] * av[4 * k + 3])
                    mbuf[e, pl.ds(j * 16, 16)] = v

        pltpu.sync_copy(mbuf, acc.at[dstv[b]], add=True)

    fire_idx(0, 0)
    wait_idx(0, 0)
    fire_gather(0)
    fire_idx(1, 1)

    def pair(q, _):
        for b in range(2):
            ci = 2 * q + b

            @pl.when(ci < nchunk)
            def _():
                wait_gather(b)

                @pl.when(ci + 1 < nchunk)
                def _():
                    wait_idx(ci + 1, 1 - b)
                    fire_gather(1 - b)

                compute_scatter(b)

                @pl.when(ci + 2 < nchunk)
                def _():
                    fire_idx(ci + 2, b)

        return 0

    lax.fori_loop(0, (NCHUNK_LAST + 1) // 2, pair, 0)

    plsc.subcore_barrier()
    pltpu.sync_copy(acc.at[rsl], out_hbm.at[cid, rsl])

    @pl.when(sid == NS - 1)
    def _():
        pltpu.sync_copy(acc.at[tsl], out_hbm.at[cid, tsl])


def _sc_edge_pass(y, ei, attr_t, zeros):
    mesh = plsc.VectorSubcoreMesh(core_axis_name="c", subcore_axis_name="s",
                                  num_cores=NC, num_subcores=NS)
    f = pl.kernel(
        _sc_edge_body,
        out_type=jax.ShapeDtypeStruct((NC, N, MW), jnp.float32),
        mesh=mesh,
        scratch_types=[
            pltpu.VMEM((C,), jnp.int32),
            pltpu.VMEM((C,), jnp.int32),
            pltpu.VMEM((C,), jnp.int32),
            pltpu.VMEM((C,), jnp.int32),
            pltpu.VMEM((C * NE,), jnp.float32),
            pltpu.VMEM((C * NE,), jnp.float32),
            pltpu.VMEM((C, NE * NOUT), jnp.float32),
            pltpu.VMEM((C, NE * NOUT), jnp.float32),
            pltpu.VMEM((C, MW), jnp.float32),
            pltpu.VMEM_SHARED((N, MW), jnp.float32),
            pltpu.SemaphoreType.DMA,
            pltpu.SemaphoreType.DMA,
            pltpu.SemaphoreType.DMA,
            pltpu.SemaphoreType.DMA,
        ],
    )
    return f(y, ei, attr_t, zeros)


def _sc_pool_body(p0_hbm, p1_hbm, b_hbm, batch_hbm, z64_hbm, ninf_hbm,
                  psum_hbm, pmax_hbm, pcnt_hbm,
                  h0, h1, bb, bv, sacc, macc, cacc):
    cid = lax.axis_index("c")
    sid = lax.axis_index("s")
    wid = sid * NC + cid

    base = wid * POOL_STEP
    nsl = pl.ds(base, POOL_BUF)
    pltpu.sync_copy(p0_hbm.at[nsl], h0)
    pltpu.sync_copy(p1_hbm.at[nsl], h1)
    pltpu.sync_copy(batch_hbm.at[nsl], bb.at[pl.ds(0, POOL_BUF)])
    pltpu.sync_copy(b_hbm, bv)
    pltpu.sync_copy(z64_hbm, sacc)
    pltpu.sync_copy(ninf_hbm, macc)
    pltpu.sync_copy(z64_hbm.at[0], cacc)

    nrows = jnp.where(wid == NW - 1, POOL_BUF, POOL_STEP)

    def node(i, _):
        b = bb[pl.ds(i, 16)][0]
        for j in range(NOUT // 16):
            sl = pl.ds(j * 16, 16)
            h = jnp.maximum(h0[i, sl] + h1[i, sl] + bv[0, sl], 0.0)
            sacc[b, sl] = sacc[b, sl] + h
            macc[b, sl] = jnp.maximum(macc[b, sl], h)
        bq = b // 16
        br = b - bq * 16
        lane = lax.iota(jnp.int32, 16)
        csl = pl.ds(bq * 16, 16)
        cacc[csl] = cacc[csl] + jnp.where(lane == br, 1.0, 0.0)
        return 0

    lax.fori_loop(0, nrows, node, 0)
    pltpu.sync_copy(sacc, psum_hbm.at[wid])
    pltpu.sync_copy(macc, pmax_hbm.at[wid])
    pltpu.sync_copy(cacc, pcnt_hbm.at[wid])


def _sc_pool(p0, p1, b2, batch, z64, ninf):
    mesh = plsc.VectorSubcoreMesh(core_axis_name="c", subcore_axis_name="s",
                                  num_cores=NC, num_subcores=NS)
    f = pl.kernel(
        _sc_pool_body,
        out_type=(jax.ShapeDtypeStruct((NW, NG, NOUT), jnp.float32),
                  jax.ShapeDtypeStruct((NW, NG, NOUT), jnp.float32),
                  jax.ShapeDtypeStruct((NW, NG), jnp.float32)),
        mesh=mesh,
        scratch_types=[
            pltpu.VMEM((POOL_BUF, MW), jnp.float32),
            pltpu.VMEM((POOL_BUF, MW), jnp.float32),
            pltpu.VMEM((POOL_BUF + 16,), jnp.int32),
            pltpu.VMEM((1, NOUT), jnp.float32),
            pltpu.VMEM((NG, NOUT), jnp.float32),
            pltpu.VMEM((NG, NOUT), jnp.float32),
            pltpu.VMEM((NG,), jnp.float32),
        ],
    )
    return f(p0, p1, b2.reshape(1, NOUT), batch, z64, ninf)


def _final_kernel(ps_ref, pm_ref, pc_ref, wfc_ref, bfc_ref, o_ref):
    s = jnp.sum(ps_ref[...], axis=0)
    mx = jnp.max(pm_ref[...], axis=0)
    cnt = jnp.sum(pc_ref[...], axis=0)
    mean = s / jnp.maximum(cnt, 1.0)[:, None]
    g = jnp.concatenate([mean, mx], axis=1)
    logits = jnp.dot(g, wfc_ref[...], preferred_element_type=jnp.float32)
    logits = logits + bfc_ref[...]
    m = jnp.max(logits, axis=1, keepdims=True)
    lse = m + jnp.log(jnp.sum(jnp.exp(logits - m), axis=1, keepdims=True))
    o_ref[...] = logits - lse


def _tc_final(psum, pmax, pcnt, wfc, bfc):
    return pl.pallas_call(
        _final_kernel,
        in_specs=[pl.BlockSpec((NW, NG, NOUT), lambda: (0, 0, 0)),
                  pl.BlockSpec((NW, NG, NOUT), lambda: (0, 0, 0)),
                  pl.BlockSpec((NW, NG), lambda: (0, 0)),
                  pl.BlockSpec((2 * NOUT, 2), lambda: (0, 0)),
                  pl.BlockSpec((1, 2), lambda: (0, 0))],
        out_specs=pl.BlockSpec((NG, 2), lambda: (0, 0)),
        out_shape=jax.ShapeDtypeStruct((NG, 2), jnp.float32),
    )(psum, pmax, pcnt, wfc, bfc.reshape(1, 2))


def kernel(x, edge_index, edge_attr, batch, W1, b1, W2, b2, Wfc, bfc):
    w1c = jnp.transpose(W1, (1, 0, 2)).reshape(NINP, NE * NOUT)
    w2c = jnp.transpose(W2, (1, 0, 2)).reshape(NOUT, NE * NOUT)
    zeros = jnp.zeros((N, MW), jnp.float32)
    ninf = jnp.full((NG, NOUT), -jnp.inf, jnp.float32)
    z64 = jnp.zeros((NG, NOUT), jnp.float32)

    ei_flat = edge_index.reshape(-1)
    attr_flat = edge_attr.reshape(-1)
    y1 = _tc_matmul(x, w1c, 2000)
    p1 = _sc_edge_pass(y1, ei_flat, attr_flat, zeros)
    y2 = _tc_combine_matmul(p1[0], p1[1], b1, w2c, 2000)
    p2 = _sc_edge_pass(y2, ei_flat, attr_flat, zeros)
    psum, pmax, pcnt = _sc_pool(p2[0], p2[1], b2, batch, z64, ninf)
    return _tc_final(psum, pmax, pcnt, Wfc, bfc)
